# baseline (speedup 1.0000x reference)
import functools
import math

import jax
import jax.numpy as jnp
from jax import lax
from jax.experimental import pallas as pl
from jax.experimental.pallas import tpu as pltpu
from jax.experimental.pallas import tpu_sc as plsc

N0, E0, D, HID, LAYERS, RATIO, OUT = 10000, 320000, 128, 128, 3, 0.8, 16
NP = 10240
EP = 327680
NC, NS = 2, 16
NW = NC * NS
EPW = EP // NW
RPT = NP // NS

_MESH = plsc.VectorSubcoreMesh(core_axis_name="c", subcore_axis_name="s")


def _zero_vmem1d(ref, n):
    z16 = jnp.zeros((16,), jnp.float32)
    for q in range(n // 16):
        ref[pl.ds(q * 16, 16)] = z16


def _zero_vmem2d(ref, r, n):
    z16 = jnp.zeros((16,), jnp.float32)
    for i in range(r):
        for q in range(n // 16):
            ref[i, pl.ds(q * 16, 16)] = z16


DH = D // 2
EB1, NBUF1 = 128, 5
NB1 = EPW // EB1
EB2, NBUF2 = 128, 8
NB2 = EPW // EB2


def _sc1_body(do_cnt, h_hbm, nv_hbm, src_hbm, dst_hbm, agg_out, cnt_out,
              sidx, didx, rows, nvv, zrow, zc, agg_s, cnt_s, *gsems):
    c = lax.axis_index("c")
    s = lax.axis_index("s")
    wid = s * NC + c

    def gather(j, b):
        pltpu.async_copy(h_hbm.at[sidx.at[j]], rows.at[b], gsems[b])
        if do_cnt:
            pltpu.async_copy(nv_hbm.at[sidx.at[j]], nvv.at[b], gsems[b])

    def gwait(j, b):
        pltpu.make_async_copy(h_hbm.at[sidx.at[j]], rows.at[b], gsems[b]).wait()
        if do_cnt:
            pltpu.make_async_copy(nv_hbm.at[sidx.at[j]], nvv.at[b],
                                  gsems[b]).wait()

    pltpu.sync_copy(src_hbm.at[wid], sidx)
    pltpu.sync_copy(dst_hbm.at[wid], didx)
    for b in range(NBUF1):
        gather(b, b)

    _zero_vmem2d(zrow, 16, DH)
    if do_cnt:
        _zero_vmem1d(zc, 128)

    def zloop(t, _):
        pltpu.sync_copy(zrow, agg_s.at[pl.ds(s * RPT + t * 16, 16)])
        return _

    lax.fori_loop(0, RPT // 16, zloop, 0)

    if do_cnt:
        def zloop2(t, _):
            pltpu.sync_copy(zc, cnt_s.at[pl.ds(s * RPT + t * 128, 128)])
            return _

        lax.fori_loop(0, RPT // 128, zloop2, 0)
    plsc.subcore_barrier()

    def step(j, b, prefetch):
        gwait(j, b)
        pltpu.sync_copy(rows.at[b], agg_s.at[didx.at[j]], add=True)
        if do_cnt:
            pltpu.sync_copy(nvv.at[b], cnt_s.at[didx.at[j]], add=True)
        if prefetch:
            gather(j + NBUF1, b)

    def grp(g, _):
        for b in range(NBUF1):
            step(g * NBUF1 + b, b, True)
        return _

    lax.fori_loop(0, NB1 // NBUF1 - 1, grp, 0)
    for b in range(NBUF1):
        step(NB1 - NBUF1 + b, b, False)

    plsc.subcore_barrier()
    pltpu.sync_copy(agg_s.at[pl.ds(s * RPT, RPT)], agg_out.at[c, pl.ds(s * RPT, RPT)])
    if do_cnt:
        pltpu.sync_copy(cnt_s.at[pl.ds(s * RPT, RPT)], cnt_out.at[c, pl.ds(s * RPT, RPT)])


def _make_sc1(do_cnt):
    outs = [jax.ShapeDtypeStruct((NC, NP, DH), jnp.float32)]
    if do_cnt:
        outs.append(jax.ShapeDtypeStruct((NC, NP), jnp.float32))
        body = functools.partial(_sc1_body, True)
    else:
        def body(h_hbm, nv_hbm, src_hbm, dst_hbm, agg_out, *rest):
            _sc1_body(False, h_hbm, nv_hbm, src_hbm, dst_hbm,
                      agg_out, None, *rest)
    return pl.kernel(
        body,
        out_type=tuple(outs) if do_cnt else outs[0],
        mesh=_MESH,
        compiler_params=pltpu.CompilerParams(use_tc_tiling_on_sc=False),
        scratch_types=[
            pltpu.VMEM((NB1, EB1), jnp.int32),
            pltpu.VMEM((NB1, EB1), jnp.int32),
            pltpu.VMEM((NBUF1, EB1, DH), jnp.float32),
            pltpu.VMEM((NBUF1, EB1), jnp.float32),
            pltpu.VMEM((16, DH), jnp.float32),
            pltpu.VMEM((128,), jnp.float32),
            pltpu.VMEM_SHARED((NP, DH), jnp.float32),
            pltpu.VMEM_SHARED((NP,), jnp.float32),
        ] + [pltpu.SemaphoreType.DMA] * NBUF1,
    )


_sc1a = _make_sc1(True)
_sc1b = _make_sc1(False)


def _sc2_body(hwd_hbm, src_hbm, dst_hbm, t_out,
              sidx, didx, vals, zc, t_s, *gsems):
    c = lax.axis_index("c")
    s = lax.axis_index("s")
    wid = s * NC + c
    pltpu.sync_copy(src_hbm.at[wid], sidx)
    pltpu.sync_copy(dst_hbm.at[wid], didx)
    for b in range(NBUF2):
        pltpu.async_copy(hwd_hbm.at[sidx.at[b]], vals.at[b], gsems[b])
    _zero_vmem1d(zc, 128)

    def zloop(t, _):
        pltpu.sync_copy(zc, t_s.at[pl.ds(s * RPT + t * 128, 128)])
        return _

    lax.fori_loop(0, RPT // 128, zloop, 0)
    plsc.subcore_barrier()

    def step(j, b, prefetch):
        pltpu.make_async_copy(hwd_hbm.at[sidx.at[j]], vals.at[b], gsems[b]).wait()
        pltpu.sync_copy(vals.at[b], t_s.at[didx.at[j]], add=True)
        if prefetch:
            pltpu.async_copy(hwd_hbm.at[sidx.at[j + NBUF2]], vals.at[b], gsems[b])

    def grp(g, _):
        for b in range(NBUF2):
            step(g * NBUF2 + b, b, True)
        return _

    lax.fori_loop(0, NB2 // NBUF2 - 1, grp, 0)
    for b in range(NBUF2):
        step(NB2 - NBUF2 + b, b, False)
    plsc.subcore_barrier()
    pltpu.sync_copy(t_s.at[pl.ds(s * RPT, RPT)], t_out.at[c, pl.ds(s * RPT, RPT)])


_sc2 = pl.kernel(
    _sc2_body,
    out_type=jax.ShapeDtypeStruct((NC, NP), jnp.float32),
    mesh=_MESH,
    compiler_params=pltpu.CompilerParams(use_tc_tiling_on_sc=False),
    scratch_types=[
        pltpu.VMEM((NB2, EB2), jnp.int32),
        pltpu.VMEM((NB2, EB2), jnp.int32),
        pltpu.VMEM((NBUF2, EB2), jnp.float32),
        pltpu.VMEM((128,), jnp.float32),
        pltpu.VMEM_SHARED((NP,), jnp.float32),
    ] + [pltpu.SemaphoreType.DMA] * NBUF2,
)


def _tc1_body(Ni, h_ref, agg_ref, cnt_ref, nv_ref, wl_ref, bl_ref, wr_ref,
              g_ref, b_ref, pw_ref, hb_ref, hw_ref, dinv_ref):
    agg = agg_ref[...]
    cnt = cnt_ref[...]
    nv = nv_ref[...]
    mean = agg / jnp.maximum(cnt, 1.0)
    hh = (jnp.dot(mean, wl_ref[...], preferred_element_type=jnp.float32)
          + bl_ref[...]
          + jnp.dot(h_ref[...], wr_ref[...], preferred_element_type=jnp.float32))
    hh = jnp.maximum(hh, 0.0) * nv * (1.0 / math.sqrt(Ni))
    mu = jnp.sum(hh, axis=0, keepdims=True) * (1.0 / Ni)
    dm = (hh - mu) * nv
    var = jnp.sum(dm * dm, axis=0, keepdims=True) * (1.0 / Ni)
    hb = ((hh - mu) * lax.rsqrt(var + 1e-5) * g_ref[...] + b_ref[...]) * nv
    hb_ref[...] = hb
    hw = jnp.dot(hb, pw_ref[...], preferred_element_type=jnp.float32)
    dinv = lax.rsqrt(cnt + 1.0) * nv
    hw_ref[...] = hw
    dinv_ref[...] = dinv


def _tc1(i, Ni, h, agg, cnt, nv, params):
    return pl.pallas_call(
        functools.partial(_tc1_body, Ni),
        out_shape=(jax.ShapeDtypeStruct((NP, D), jnp.float32),
                   jax.ShapeDtypeStruct((NP, 1), jnp.float32),
                   jax.ShapeDtypeStruct((NP, 1), jnp.float32)),
    )(h, agg, cnt, nv, params[f"Wl{i}"], params[f"bl{i}"][None, :],
      params[f"Wr{i}"], params[f"bn_g{i}"][None, :], params[f"bn_b{i}"][None, :],
      params[f"pW{i}"])


def _tc2a_body(k, hw_ref, dinv_ref, t_ref, pb_ref, gate_ref, nv_ref):
    hw = hw_ref[...]
    dinv = dinv_ref[...]
    score = dinv * t_ref[...] + dinv * dinv * hw + pb_ref[0, 0]
    score_m = jnp.where(dinv > 0, score, -3.4e38)
    b32 = lax.bitcast_convert_type(score_m, jnp.int32)
    u = lax.bitcast_convert_type(score_m, jnp.uint32)
    srt = jnp.where(b32 < 0, ~u, u | jnp.uint32(0x80000000))

    def bit_step(j, pref):
        cand = pref | (jnp.uint32(1) << (31 - j).astype(jnp.uint32))
        cge = jnp.sum((srt >= cand).astype(jnp.int32))
        return jnp.where(cge >= k, cand, pref)

    T = lax.fori_loop(0, 32, bit_step, jnp.uint32(0))
    c_gt = jnp.sum((srt > T).astype(jnp.int32))
    m = k - c_gt
    eq = srt == T
    idx = (lax.broadcasted_iota(jnp.int32, (NP // 128, 128), 0) * 128
           + lax.broadcasted_iota(jnp.int32, (NP // 128, 128), 1))

    def jstep(j, J):
        cand = J | (jnp.int32(1) << (13 - j))
        clt = jnp.sum((eq & (idx < cand)).astype(jnp.int32))
        return jnp.where(clt < m, cand, J)

    J = lax.fori_loop(0, 14, jstep, jnp.int32(0))
    kept = (srt > T) | (eq & (idx <= J))
    nvn = kept.astype(jnp.float32)
    nv_ref[...] = nvn
    gate_ref[...] = jnp.tanh(score) * nvn


def _tc2a(k, hw80, dinv80, t80, pb):
    return pl.pallas_call(
        functools.partial(_tc2a_body, k),
        out_shape=(jax.ShapeDtypeStruct((NP // 128, 128), jnp.float32),
                   jax.ShapeDtypeStruct((NP // 128, 128), jnp.float32)),
    )(hw80, dinv80, t80, pb)


def _tc2b_body(k, hb_ref, gate_ref, nv_ref, xs_ref, hn_ref, xs_out_ref):
    hn = hb_ref[...] * gate_ref[...]
    hn_ref[...] = hn
    gmean = jnp.sum(hn, axis=0, keepdims=True) * (1.0 / k)
    gmax = jnp.max(jnp.where(nv_ref[...] > 0, hn, -3.4e38), axis=0, keepdims=True)
    xs_out_ref[...] = xs_ref[...] + jnp.concatenate([gmean, gmax], axis=1)


def _tc2b(k, hb, gate, nv, xs):
    return pl.pallas_call(
        functools.partial(_tc2b_body, k),
        out_shape=(jax.ShapeDtypeStruct((NP, D), jnp.float32),
                   jax.ShapeDtypeStruct((1, 2 * HID), jnp.float32)),
    )(hb, gate, nv, xs)


def _tc2b_last_body(k, hb_ref, gate_ref, nv_ref, xs_ref,
                    w1_ref, b1_ref, w2_ref, b2_ref, w3_ref, b3_ref, o_ref):
    hn = hb_ref[...] * gate_ref[...]
    gmean = jnp.sum(hn, axis=0, keepdims=True) * (1.0 / k)
    gmax = jnp.max(jnp.where(nv_ref[...] > 0, hn, -3.4e38), axis=0, keepdims=True)
    xs = xs_ref[...] + jnp.concatenate([gmean, gmax], axis=1)
    o = jnp.maximum(jnp.dot(xs, w1_ref[...], preferred_element_type=jnp.float32) + b1_ref[...], 0.0)
    o = jnp.maximum(jnp.dot(o, w2_ref[...], preferred_element_type=jnp.float32) + b2_ref[...], 0.0)
    o_ref[...] = jnp.dot(o, w3_ref[...], preferred_element_type=jnp.float32) + b3_ref[...]


def _tc2b_last(k, hb, gate, nv, xs, params):
    return pl.pallas_call(
        functools.partial(_tc2b_last_body, k),
        out_shape=jax.ShapeDtypeStruct((1, OUT), jnp.float32),
    )(hb, gate, nv, xs, params["lin1_W"], params["lin1_b"][None, :],
      params["lin2_W"], params["lin2_b"][None, :],
      params["lin3_W"], params["lin3_b"][None, :])


def kernel(x, params, edge_index, batch):
    src = edge_index[0]
    dst = edge_index[1]
    pad = (jnp.arange(EP - E0, dtype=jnp.int32) % (NP - N0)) + N0
    srcp = jnp.concatenate([src, pad])
    dstp = jnp.concatenate([dst, pad])
    src1 = srcp.reshape(NW, NB1, EB1)
    dst1 = dstp.reshape(NW, NB1, EB1)
    src2 = srcp.reshape(NW, NB2, EB2)
    dst2 = dstp.reshape(NW, NB2, EB2)
    h = jnp.zeros((NP, D), jnp.float32).at[:N0].set(x)
    nv_flat = (jnp.arange(NP) < N0).astype(jnp.float32)
    nv_col = nv_flat.reshape(NP, 1)
    Ns_l = [N0, int(math.ceil(RATIO * N0)),
            int(math.ceil(RATIO * math.ceil(RATIO * N0)))]
    xs = jnp.zeros((1, 2 * HID), jnp.float32)
    out = None
    for i in range(LAYERS):
        Ni = Ns_l[i]
        k = int(math.ceil(RATIO * Ni))
        agga, cnt2 = _sc1a(h[:, :DH], nv_flat, src1, dst1)
        aggb = _sc1b(h[:, DH:], nv_flat, src1, dst1)
        agg = jnp.concatenate([agga[0] + agga[1], aggb[0] + aggb[1]], axis=1)
        cnt = (cnt2[0] + cnt2[1]).reshape(NP, 1)
        hb, hw, dinv = _tc1(i, Ni, h, agg, cnt, nv_col, params)
        hwd = (hw * dinv).reshape(NP)
        t2 = _sc2(hwd, src2, dst2)
        t80 = (t2[0] + t2[1]).reshape(NP // 128, 128)
        gate80, nv80 = _tc2a(k, hw.reshape(NP // 128, 128),
                             dinv.reshape(NP // 128, 128), t80,
                             params[f"pb{i}"].reshape(1, 1))
        gate = gate80.reshape(NP, 1)
        nv_col = nv80.reshape(NP, 1)
        nv_flat = nv80.reshape(NP)
        if i < LAYERS - 1:
            h, xs = _tc2b(k, hb, gate, nv_col, xs)
        else:
            out = _tc2b_last(k, hb, gate, nv_col, xs, params)
    return out


# speedup vs baseline: 1.1348x; 1.1348x over previous
import functools
import math

import jax
import jax.numpy as jnp
from jax import lax
from jax.experimental import pallas as pl
from jax.experimental.pallas import tpu as pltpu
from jax.experimental.pallas import tpu_sc as plsc

N0, E0, D, HID, LAYERS, RATIO, OUT = 10000, 320000, 128, 128, 3, 0.8, 16
NP = 10240
EP = 327680
NC, NS = 2, 16
NW = NC * NS
EPW = EP // NW
RPT = NP // NS

_MESH = plsc.VectorSubcoreMesh(core_axis_name="c", subcore_axis_name="s")


def _zero_vmem1d(ref, n):
    z16 = jnp.zeros((16,), jnp.float32)
    for q in range(n // 16):
        ref[pl.ds(q * 16, 16)] = z16


def _zero_vmem2d(ref, r, n):
    z16 = jnp.zeros((16,), jnp.float32)
    for i in range(r):
        for q in range(n // 16):
            ref[i, pl.ds(q * 16, 16)] = z16


DH = D // 2
EB1, NBUF1 = 128, 5
NB1 = EPW // EB1
EB2, NBUF2 = 128, 8
NB2 = EPW // EB2


def _sc1_body(do_cnt, h_hbm, nv_hbm, src_hbm, dst_hbm, agg_out, cnt_out,
              sidx, didx, rows, nvv, zrow, zc, agg_s, cnt_s, *gsems):
    c = lax.axis_index("c")
    s = lax.axis_index("s")
    wid = s * NC + c

    def gather(j, b):
        pltpu.async_copy(h_hbm.at[sidx.at[j]], rows.at[b], gsems[b])
        if do_cnt:
            pltpu.async_copy(nv_hbm.at[sidx.at[j]], nvv.at[b], gsems[b])

    def gwait(j, b):
        pltpu.make_async_copy(h_hbm.at[sidx.at[j]], rows.at[b], gsems[b]).wait()
        if do_cnt:
            pltpu.make_async_copy(nv_hbm.at[sidx.at[j]], nvv.at[b],
                                  gsems[b]).wait()

    pltpu.sync_copy(src_hbm.at[wid], sidx)
    pltpu.sync_copy(dst_hbm.at[wid], didx)
    for b in range(NBUF1):
        gather(b, b)

    _zero_vmem2d(zrow, 16, DH)
    if do_cnt:
        _zero_vmem1d(zc, 128)

    def zloop(t, _):
        pltpu.sync_copy(zrow, agg_s.at[pl.ds(s * RPT + t * 16, 16)])
        return _

    lax.fori_loop(0, RPT // 16, zloop, 0)

    if do_cnt:
        def zloop2(t, _):
            pltpu.sync_copy(zc, cnt_s.at[pl.ds(s * RPT + t * 128, 128)])
            return _

        lax.fori_loop(0, RPT // 128, zloop2, 0)
    plsc.subcore_barrier()

    def step(j, b, prefetch):
        gwait(j, b)
        pltpu.sync_copy(rows.at[b], agg_s.at[didx.at[j]], add=True)
        if do_cnt:
            pltpu.sync_copy(nvv.at[b], cnt_s.at[didx.at[j]], add=True)
        if prefetch:
            gather(j + NBUF1, b)

    def grp(g, _):
        for b in range(NBUF1):
            step(g * NBUF1 + b, b, True)
        return _

    lax.fori_loop(0, NB1 // NBUF1 - 1, grp, 0)
    for b in range(NBUF1):
        step(NB1 - NBUF1 + b, b, False)

    plsc.subcore_barrier()
    pltpu.sync_copy(agg_s.at[pl.ds(s * RPT, RPT)], agg_out.at[c, pl.ds(s * RPT, RPT)])
    if do_cnt:
        pltpu.sync_copy(cnt_s.at[pl.ds(s * RPT, RPT)], cnt_out.at[c, pl.ds(s * RPT, RPT)])


def _make_sc1(do_cnt):
    outs = [jax.ShapeDtypeStruct((NC, NP, DH), jnp.float32)]
    if do_cnt:
        outs.append(jax.ShapeDtypeStruct((NC, NP), jnp.float32))
        body = functools.partial(_sc1_body, True)
    else:
        def body(h_hbm, nv_hbm, src_hbm, dst_hbm, agg_out, *rest):
            _sc1_body(False, h_hbm, nv_hbm, src_hbm, dst_hbm,
                      agg_out, None, *rest)
    return pl.kernel(
        body,
        out_type=tuple(outs) if do_cnt else outs[0],
        mesh=_MESH,
        compiler_params=pltpu.CompilerParams(use_tc_tiling_on_sc=False),
        scratch_types=[
            pltpu.VMEM((NB1, EB1), jnp.int32),
            pltpu.VMEM((NB1, EB1), jnp.int32),
            pltpu.VMEM((NBUF1, EB1, DH), jnp.float32),
            pltpu.VMEM((NBUF1, EB1), jnp.float32),
            pltpu.VMEM((16, DH), jnp.float32),
            pltpu.VMEM((128,), jnp.float32),
            pltpu.VMEM_SHARED((NP, DH), jnp.float32),
            pltpu.VMEM_SHARED((NP,), jnp.float32),
        ] + [pltpu.SemaphoreType.DMA] * NBUF1,
    )


_sc1a = _make_sc1(True)
_sc1b = _make_sc1(False)


def _sc2_body(hwd_hbm, src_hbm, dst_hbm, t_out,
              hwd_v, sidx, didx, vals, zc, t_s, *ssems):
    c = lax.axis_index("c")
    s = lax.axis_index("s")
    wid = s * NC + c
    pltpu.sync_copy(hwd_hbm, hwd_v)
    pltpu.sync_copy(src_hbm.at[wid], sidx)
    pltpu.sync_copy(dst_hbm.at[wid], didx)
    _zero_vmem1d(zc, 128)

    def zloop(t, _):
        pltpu.sync_copy(zc, t_s.at[pl.ds(s * RPT + t * 128, 128)])
        return _

    lax.fori_loop(0, RPT // 128, zloop, 0)
    plsc.subcore_barrier()

    def fill(j, b):
        def q_step(q, _):
            idxv = sidx[j, pl.ds(q * 16, 16)]
            vals[b, pl.ds(q * 16, 16)] = plsc.load_gather(hwd_v, [idxv])
            return _

        lax.fori_loop(0, EB2 // 16, q_step, 0)

    def scat(j, b):
        pltpu.async_copy(vals.at[b], t_s.at[didx.at[j]], ssems[b], add=True)

    def swait(j, b):
        pltpu.make_async_copy(vals.at[b], t_s.at[didx.at[j]], ssems[b]).wait()

    for j in range(2):
        fill(j, j)
        scat(j, j)

    def grp(g, _):
        for b in range(2):
            j = g * 2 + b
            swait(j - 2, b)
            fill(j, b)
            scat(j, b)
        return _

    lax.fori_loop(1, NB2 // 2, grp, 0)
    for j in range(NB2 - 2, NB2):
        swait(j, j % 2)
    plsc.subcore_barrier()
    pltpu.sync_copy(t_s.at[pl.ds(s * RPT, RPT)], t_out.at[c, pl.ds(s * RPT, RPT)])


_sc2 = pl.kernel(
    _sc2_body,
    out_type=jax.ShapeDtypeStruct((NC, NP), jnp.float32),
    mesh=_MESH,
    compiler_params=pltpu.CompilerParams(use_tc_tiling_on_sc=False, needs_layout_passes=False),
    scratch_types=[
        pltpu.VMEM((NP,), jnp.float32),
        pltpu.VMEM((NB2, EB2), jnp.int32),
        pltpu.VMEM((NB2, EB2), jnp.int32),
        pltpu.VMEM((2, EB2), jnp.float32),
        pltpu.VMEM((128,), jnp.float32),
        pltpu.VMEM_SHARED((NP,), jnp.float32),
        pltpu.SemaphoreType.DMA,
        pltpu.SemaphoreType.DMA,
    ],
)


def _tc1_body(Ni, h_ref, agg_ref, cnt_ref, nv_ref, wl_ref, bl_ref, wr_ref,
              g_ref, b_ref, pw_ref, hb_ref, hw_ref, dinv_ref):
    agg = agg_ref[...]
    cnt = cnt_ref[...]
    nv = nv_ref[...]
    mean = agg / jnp.maximum(cnt, 1.0)
    hh = (jnp.dot(mean, wl_ref[...], preferred_element_type=jnp.float32)
          + bl_ref[...]
          + jnp.dot(h_ref[...], wr_ref[...], preferred_element_type=jnp.float32))
    hh = jnp.maximum(hh, 0.0) * nv * (1.0 / math.sqrt(Ni))
    mu = jnp.sum(hh, axis=0, keepdims=True) * (1.0 / Ni)
    dm = (hh - mu) * nv
    var = jnp.sum(dm * dm, axis=0, keepdims=True) * (1.0 / Ni)
    hb = ((hh - mu) * lax.rsqrt(var + 1e-5) * g_ref[...] + b_ref[...]) * nv
    hb_ref[...] = hb
    hw = jnp.dot(hb, pw_ref[...], preferred_element_type=jnp.float32)
    dinv = lax.rsqrt(cnt + 1.0) * nv
    hw_ref[...] = hw
    dinv_ref[...] = dinv


def _tc1(i, Ni, h, agg, cnt, nv, params):
    return pl.pallas_call(
        functools.partial(_tc1_body, Ni),
        out_shape=(jax.ShapeDtypeStruct((NP, D), jnp.float32),
                   jax.ShapeDtypeStruct((NP, 1), jnp.float32),
                   jax.ShapeDtypeStruct((NP, 1), jnp.float32)),
    )(h, agg, cnt, nv, params[f"Wl{i}"], params[f"bl{i}"][None, :],
      params[f"Wr{i}"], params[f"bn_g{i}"][None, :], params[f"bn_b{i}"][None, :],
      params[f"pW{i}"])


def _tc2a_body(k, hw_ref, dinv_ref, t_ref, pb_ref, gate_ref, nv_ref):
    hw = hw_ref[...]
    dinv = dinv_ref[...]
    score = dinv * t_ref[...] + dinv * dinv * hw + pb_ref[0, 0]
    score_m = jnp.where(dinv > 0, score, -3.4e38)
    b32 = lax.bitcast_convert_type(score_m, jnp.int32)
    u = lax.bitcast_convert_type(score_m, jnp.uint32)
    srt = jnp.where(b32 < 0, ~u, u | jnp.uint32(0x80000000))

    def bit_step(j, pref):
        cand = pref | (jnp.uint32(1) << (31 - j).astype(jnp.uint32))
        cge = jnp.sum((srt >= cand).astype(jnp.int32))
        return jnp.where(cge >= k, cand, pref)

    T = lax.fori_loop(0, 32, bit_step, jnp.uint32(0))
    c_gt = jnp.sum((srt > T).astype(jnp.int32))
    m = k - c_gt
    eq = srt == T
    idx = (lax.broadcasted_iota(jnp.int32, (NP // 128, 128), 0) * 128
           + lax.broadcasted_iota(jnp.int32, (NP // 128, 128), 1))

    def jstep(j, J):
        cand = J | (jnp.int32(1) << (13 - j))
        clt = jnp.sum((eq & (idx < cand)).astype(jnp.int32))
        return jnp.where(clt < m, cand, J)

    J = lax.fori_loop(0, 14, jstep, jnp.int32(0))
    kept = (srt > T) | (eq & (idx <= J))
    nvn = kept.astype(jnp.float32)
    nv_ref[...] = nvn
    gate_ref[...] = jnp.tanh(score) * nvn


def _tc2a(k, hw80, dinv80, t80, pb):
    return pl.pallas_call(
        functools.partial(_tc2a_body, k),
        out_shape=(jax.ShapeDtypeStruct((NP // 128, 128), jnp.float32),
                   jax.ShapeDtypeStruct((NP // 128, 128), jnp.float32)),
    )(hw80, dinv80, t80, pb)


def _tc2b_body(k, hb_ref, gate_ref, nv_ref, xs_ref, hn_ref, xs_out_ref):
    hn = hb_ref[...] * gate_ref[...]
    hn_ref[...] = hn
    gmean = jnp.sum(hn, axis=0, keepdims=True) * (1.0 / k)
    gmax = jnp.max(jnp.where(nv_ref[...] > 0, hn, -3.4e38), axis=0, keepdims=True)
    xs_out_ref[...] = xs_ref[...] + jnp.concatenate([gmean, gmax], axis=1)


def _tc2b(k, hb, gate, nv, xs):
    return pl.pallas_call(
        functools.partial(_tc2b_body, k),
        out_shape=(jax.ShapeDtypeStruct((NP, D), jnp.float32),
                   jax.ShapeDtypeStruct((1, 2 * HID), jnp.float32)),
    )(hb, gate, nv, xs)


def _tc2b_last_body(k, hb_ref, gate_ref, nv_ref, xs_ref,
                    w1_ref, b1_ref, w2_ref, b2_ref, w3_ref, b3_ref, o_ref):
    hn = hb_ref[...] * gate_ref[...]
    gmean = jnp.sum(hn, axis=0, keepdims=True) * (1.0 / k)
    gmax = jnp.max(jnp.where(nv_ref[...] > 0, hn, -3.4e38), axis=0, keepdims=True)
    xs = xs_ref[...] + jnp.concatenate([gmean, gmax], axis=1)
    o = jnp.maximum(jnp.dot(xs, w1_ref[...], preferred_element_type=jnp.float32) + b1_ref[...], 0.0)
    o = jnp.maximum(jnp.dot(o, w2_ref[...], preferred_element_type=jnp.float32) + b2_ref[...], 0.0)
    o_ref[...] = jnp.dot(o, w3_ref[...], preferred_element_type=jnp.float32) + b3_ref[...]


def _tc2b_last(k, hb, gate, nv, xs, params):
    return pl.pallas_call(
        functools.partial(_tc2b_last_body, k),
        out_shape=jax.ShapeDtypeStruct((1, OUT), jnp.float32),
    )(hb, gate, nv, xs, params["lin1_W"], params["lin1_b"][None, :],
      params["lin2_W"], params["lin2_b"][None, :],
      params["lin3_W"], params["lin3_b"][None, :])


def kernel(x, params, edge_index, batch):
    src = edge_index[0]
    dst = edge_index[1]
    pad = (jnp.arange(EP - E0, dtype=jnp.int32) % (NP - N0)) + N0
    srcp = jnp.concatenate([src, pad])
    dstp = jnp.concatenate([dst, pad])
    src1 = srcp.reshape(NW, NB1, EB1)
    dst1 = dstp.reshape(NW, NB1, EB1)
    src2 = srcp.reshape(NW, NB2, EB2)
    dst2 = dstp.reshape(NW, NB2, EB2)
    h = jnp.zeros((NP, D), jnp.float32).at[:N0].set(x)
    nv_flat = (jnp.arange(NP) < N0).astype(jnp.float32)
    nv_col = nv_flat.reshape(NP, 1)
    Ns_l = [N0, int(math.ceil(RATIO * N0)),
            int(math.ceil(RATIO * math.ceil(RATIO * N0)))]
    xs = jnp.zeros((1, 2 * HID), jnp.float32)
    out = None
    for i in range(LAYERS):
        Ni = Ns_l[i]
        k = int(math.ceil(RATIO * Ni))
        agga, cnt2 = _sc1a(h[:, :DH], nv_flat, src1, dst1)
        aggb = _sc1b(h[:, DH:], nv_flat, src1, dst1)
        agg = jnp.concatenate([agga[0] + agga[1], aggb[0] + aggb[1]], axis=1)
        cnt = (cnt2[0] + cnt2[1]).reshape(NP, 1)
        hb, hw, dinv = _tc1(i, Ni, h, agg, cnt, nv_col, params)
        hwd = (hw * dinv).reshape(NP)
        t2 = _sc2(hwd, src2, dst2)
        t80 = (t2[0] + t2[1]).reshape(NP // 128, 128)
        gate80, nv80 = _tc2a(k, hw.reshape(NP // 128, 128),
                             dinv.reshape(NP // 128, 128), t80,
                             params[f"pb{i}"].reshape(1, 1))
        gate = gate80.reshape(NP, 1)
        nv_col = nv80.reshape(NP, 1)
        nv_flat = nv80.reshape(NP)
        if i < LAYERS - 1:
            h, xs = _tc2b(k, hb, gate, nv_col, xs)
        else:
            out = _tc2b_last(k, hb, gate, nv_col, xs, params)
    return out
